# Initial kernel scaffold; baseline (speedup 1.0000x reference)
import jax
import jax.numpy as jnp
from jax.experimental import pallas as pl


def kernel(x, edge_index, W1_nei, W1_root, b1, W2_nei, W2_root, b2):
    raise NotImplementedError("write your pallas kernel here")


# speedup vs baseline: 7.4296x; 7.4296x over previous
import functools

import jax
import jax.numpy as jnp
from jax import lax
from jax.experimental import pallas as pl
from jax.experimental.pallas import tpu as pltpu
from jax.experimental.pallas import tpu_sc as plsc

N = 10000
E = 320000
D_IN = 128
D_HID = 64
D_OUT = 128

NC = 2
NS = 16
NW = NC * NS
CHUNK = 128
K = -(-E // (NW * CHUNK))
E_PAD = NW * K * CHUNK
DUMMY = N
RPT = 640
N_ACC = NS * RPT


def _make_sc_seg(with_count: bool):
    mesh = plsc.VectorSubcoreMesh(core_axis_name="c", subcore_axis_name="s")
    acc_t = jax.ShapeDtypeStruct((NC, N_ACC, D_HID), jnp.float32)
    if with_count:
        out_type = (acc_t, jax.ShapeDtypeStruct((NC, N_ACC), jnp.float32))
    else:
        out_type = acc_t

    def body(table_hbm, src_hbm, dst_hbm, zrow_hbm, z1_hbm, ones_hbm,
             *rest):
        if with_count:
            out_hbm, cnt_hbm = rest[0], rest[1]
            scr = rest[2:]
        else:
            out_hbm = rest[0]
            scr = rest[1:]
        (srcv, dstv, rows, wbuf, wbuf1, onesv, acc_sh, cnt_sh) = scr
        cid = lax.axis_index("c")
        sid = lax.axis_index("s")
        wid = sid * NC + cid

        pltpu.sync_copy(zrow_hbm, wbuf)
        pltpu.sync_copy(wbuf, acc_sh.at[pl.ds(sid * RPT, RPT)])
        if with_count:
            pltpu.sync_copy(z1_hbm, wbuf1)
            pltpu.sync_copy(wbuf1, cnt_sh.at[pl.ds(sid * RPT, RPT)])
        pltpu.sync_copy(src_hbm.at[wid], srcv)
        pltpu.sync_copy(dst_hbm.at[wid], dstv)
        if with_count:
            pltpu.sync_copy(ones_hbm, onesv)
        plsc.subcore_barrier()

        def step(j, carry):
            pltpu.sync_copy(table_hbm.at[srcv.at[j]], rows)
            pltpu.sync_copy(rows, acc_sh.at[dstv.at[j]], add=True)
            if with_count:
                pltpu.sync_copy(onesv, cnt_sh.at[dstv.at[j]], add=True)
            return carry

        lax.fori_loop(0, K, step, 0)
        plsc.subcore_barrier()

        pltpu.sync_copy(acc_sh.at[pl.ds(sid * RPT, RPT)], wbuf)
        pltpu.sync_copy(wbuf, out_hbm.at[cid, pl.ds(sid * RPT, RPT)])
        if with_count:
            pltpu.sync_copy(cnt_sh.at[pl.ds(sid * RPT, RPT)], wbuf1)
            pltpu.sync_copy(wbuf1, cnt_hbm.at[cid, pl.ds(sid * RPT, RPT)])

    return pl.kernel(
        body,
        mesh=mesh,
        out_type=out_type,
        compiler_params=pltpu.CompilerParams(use_tc_tiling_on_sc=False),
        scratch_types=[
            pltpu.VMEM((K, CHUNK), jnp.int32),
            pltpu.VMEM((K, CHUNK), jnp.int32),
            pltpu.VMEM((CHUNK, D_HID), jnp.float32),
            pltpu.VMEM((RPT, D_HID), jnp.float32),
            pltpu.VMEM((RPT,), jnp.float32),
            pltpu.VMEM((CHUNK,), jnp.float32),
            pltpu.VMEM_SHARED((N_ACC, D_HID), jnp.float32),
            pltpu.VMEM_SHARED((N_ACC,), jnp.float32),
        ],
    )


_sc_seg_cnt = _make_sc_seg(True)
_sc_seg = _make_sc_seg(False)


def _dense_a_body(x_ref, wn_ref, wr_ref, b_ref, xa_ref, xr_ref):
    x = x_ref[...]
    xa_ref[...] = jnp.dot(x, wn_ref[...], preferred_element_type=jnp.float32)
    xr_ref[...] = (jnp.dot(x, wr_ref[...], preferred_element_type=jnp.float32)
                   + b_ref[...])


def _dense_a(x, wn, wr, b):
    return pl.pallas_call(
        _dense_a_body,
        out_shape=(jax.ShapeDtypeStruct((N, D_HID), jnp.float32),
                   jax.ShapeDtypeStruct((N, D_HID), jnp.float32)),
    )(x, wn, wr, b)


def _dense_b_body(p0_ref, p1_ref, c0_ref, c1_ref, xr_ref, h_ref, rinv_ref):
    cnt = jnp.maximum(c0_ref[...] + c1_ref[...], 1.0)
    rinv = 1.0 / cnt
    rinv_ref[...] = rinv
    h_ref[...] = jnp.maximum(
        (p0_ref[...] + p1_ref[...]) * rinv + xr_ref[...], 0.0)


def _dense_b(p0, p1, c0, c1, xr):
    return pl.pallas_call(
        _dense_b_body,
        out_shape=(jax.ShapeDtypeStruct((N, D_HID), jnp.float32),
                   jax.ShapeDtypeStruct((N, 1), jnp.float32)),
    )(p0, p1, c0, c1, xr)


def _dense_c_body(q0_ref, q1_ref, rinv_ref, h_ref, wn_ref, wr_ref, b_ref,
                  out_ref):
    mean2 = (q0_ref[...] + q1_ref[...]) * rinv_ref[...]
    z = (jnp.dot(mean2, wn_ref[...], preferred_element_type=jnp.float32)
         + jnp.dot(h_ref[...], wr_ref[...], preferred_element_type=jnp.float32)
         + b_ref[...])
    z = jnp.maximum(z, 0.0)
    z = z - jnp.max(z, axis=1, keepdims=True)
    out_ref[...] = z - jnp.log(jnp.sum(jnp.exp(z), axis=1, keepdims=True))


def _dense_c(q0, q1, rinv, h, wn, wr, b):
    return pl.pallas_call(
        _dense_c_body,
        out_shape=jax.ShapeDtypeStruct((N, D_OUT), jnp.float32),
    )(q0, q1, rinv, h, wn, wr, b)


def kernel(x, edge_index, W1_nei, W1_root, b1, W2_nei, W2_root, b2):
    src = edge_index[0].astype(jnp.int32)
    dst = edge_index[1].astype(jnp.int32)
    pad = E_PAD - E
    src_p = jnp.concatenate([src, jnp.zeros((pad,), jnp.int32)]
                            ).reshape(NW, K, CHUNK)
    dst_p = jnp.concatenate([dst, jnp.full((pad,), DUMMY, jnp.int32)]
                            ).reshape(NW, K, CHUNK)
    zrow = jnp.zeros((RPT, D_HID), jnp.float32)
    z1 = jnp.zeros((RPT,), jnp.float32)
    ones_c = jnp.ones((CHUNK,), jnp.float32)

    xa, xr = _dense_a(x, W1_nei, W1_root, b1.reshape(1, D_HID))
    parts, cnts = _sc_seg_cnt(xa, src_p, dst_p, zrow, z1, ones_c)
    h, rinv = _dense_b(parts[0, :N], parts[1, :N],
                       cnts[0, :N, None], cnts[1, :N, None], xr)
    parts2 = _sc_seg(h, src_p, dst_p, zrow, z1, ones_c)
    out = _dense_c(parts2[0, :N], parts2[1, :N], rinv, h,
                   W2_nei, W2_root, b2.reshape(1, D_OUT))
    return out
